# Initial kernel scaffold; baseline (speedup 1.0000x reference)
import jax
import jax.numpy as jnp
from jax.experimental import pallas as pl


def kernel(features, idx):
    raise NotImplementedError("write your pallas kernel here")


# speedup vs baseline: 2.9658x; 2.9658x over previous
import functools

import jax
import jax.numpy as jnp
from jax import lax
from jax.experimental import pallas as pl
from jax.experimental.pallas import tpu as pltpu
from jax.experimental.pallas import tpu_sc as plsc

_L = 16


def _gather_kernel(B, C, N, M):
    NW = 32
    pairs_per_w = (B * C) // NW
    n_vec = M // _L
    _UNROLL = 4
    mesh = plsc.VectorSubcoreMesh(core_axis_name="c", subcore_axis_name="s")

    @functools.partial(
        pl.kernel,
        mesh=mesh,
        compiler_params=pltpu.CompilerParams(needs_layout_passes=False),
        out_type=jax.ShapeDtypeStruct((B * C, M), jnp.float32),
        scratch_types=[
            pltpu.VMEM((M,), jnp.int32),
            pltpu.VMEM((N,), jnp.float32),
            pltpu.VMEM((M,), jnp.float32),
        ],
    )
    def k(feat_hbm, idx_hbm, out_hbm, idx_v, frow_v, gath_v):
        wid = lax.axis_index("s") * 2 + lax.axis_index("c")
        b = wid // (NW // B)
        row0 = wid * pairs_per_w
        pltpu.sync_copy(idx_hbm.at[b], idx_v)

        def body(ci, _):
            bc = row0 + ci
            pltpu.sync_copy(feat_hbm.at[bc], frow_v)

            def gather(t, _):
                for u in range(_UNROLL):
                    off = (t * _UNROLL + u) * _L
                    iv = idx_v[pl.ds(off, _L)]
                    gath_v[pl.ds(off, _L)] = plsc.load_gather(frow_v, [iv])
                return 0

            lax.fori_loop(0, n_vec // _UNROLL, gather, 0)
            pltpu.sync_copy(gath_v, out_hbm.at[bc])
            return 0

        lax.fori_loop(0, pairs_per_w, body, 0)

    return k


def kernel(features, idx):
    B, C, N = features.shape
    M = idx.shape[1]
    feat2 = features.reshape(B * C, N)
    idx2 = idx.astype(jnp.int32)
    out = _gather_kernel(B, C, N, M)(feat2, idx2)
    return out.reshape(B, C, M)


# speedup vs baseline: 4.3921x; 1.4809x over previous
import functools

import jax
import jax.numpy as jnp
from jax import lax
from jax.experimental import pallas as pl
from jax.experimental.pallas import tpu as pltpu
from jax.experimental.pallas import tpu_sc as plsc

_L = 16
_Q = 4096


def _gather_kernel(B, C, N, M):
    NW = 32
    pairs_per_w = (B * C) // NW
    n_q = M // _Q
    _UNROLL = 8
    mesh = plsc.VectorSubcoreMesh(core_axis_name="c", subcore_axis_name="s")

    @functools.partial(
        pl.kernel,
        mesh=mesh,
        compiler_params=pltpu.CompilerParams(needs_layout_passes=False),
        out_type=jax.ShapeDtypeStruct((B * C, M), jnp.float32),
        scratch_types=[
            pltpu.VMEM((M,), jnp.int32),
            pltpu.VMEM((N,), jnp.float32),
            pltpu.VMEM((N,), jnp.float32),
            pltpu.VMEM((_Q,), jnp.float32),
            pltpu.VMEM((_Q,), jnp.float32),
            pltpu.SemaphoreType.DMA,
            pltpu.SemaphoreType.DMA,
            pltpu.SemaphoreType.DMA,
        ],
    )
    def k(feat_hbm, idx_hbm, out_hbm, idx_v, frow_a, frow_b, gath_a, gath_b,
          sem_a, sem_b, sem_o):
        wid = lax.axis_index("s") * 2 + lax.axis_index("c")
        b = wid // (NW // B)
        row0 = wid * pairs_per_w
        pltpu.sync_copy(idx_hbm.at[b], idx_v)
        pltpu.async_copy(feat_hbm.at[row0], frow_a, sem_a)

        def process(frow, bc):
            for q in range(n_q):
                gath = gath_a if q % 2 == 0 else gath_b
                if q >= 2:
                    pltpu.make_async_copy(
                        gath, out_hbm.at[bc, pl.ds((q - 2) * _Q, _Q)],
                        sem_o).wait()

                def g(t, _, q=q, gath=gath):
                    for u in range(_UNROLL):
                        off = (t * _UNROLL + u) * _L
                        iv = idx_v[pl.ds(q * _Q + off, _L)]
                        gath[pl.ds(off, _L)] = plsc.load_gather(frow, [iv])
                    return 0

                lax.fori_loop(0, _Q // (_L * _UNROLL), g, 0)
                pltpu.async_copy(gath, out_hbm.at[bc, pl.ds(q * _Q, _Q)], sem_o)
            for q in (n_q - 2, n_q - 1):
                gath = gath_a if q % 2 == 0 else gath_b
                pltpu.make_async_copy(
                    gath, out_hbm.at[bc, pl.ds(q * _Q, _Q)], sem_o).wait()

        def body(k2, _):
            bc0 = row0 + 2 * k2
            pltpu.async_copy(feat_hbm.at[bc0 + 1], frow_b, sem_b)
            pltpu.make_async_copy(feat_hbm.at[bc0], frow_a, sem_a).wait()
            process(frow_a, bc0)

            @pl.when(k2 < pairs_per_w // 2 - 1)
            def _():
                pltpu.async_copy(feat_hbm.at[bc0 + 2], frow_a, sem_a)

            pltpu.make_async_copy(feat_hbm.at[bc0 + 1], frow_b, sem_b).wait()
            process(frow_b, bc0 + 1)
            return 0

        lax.fori_loop(0, pairs_per_w // 2, body, 0)

    return k


def kernel(features, idx):
    B, C, N = features.shape
    M = idx.shape[1]
    feat2 = features.reshape(B * C, N)
    idx2 = idx.astype(jnp.int32)
    out = _gather_kernel(B, C, N, M)(feat2, idx2)
    return out.reshape(B, C, M)
